# baseline (speedup 1.0000x reference)
import jax
import jax.numpy as jnp
from jax import lax
from jax.experimental import pallas as pl
from jax.experimental.pallas import tpu as pltpu
from jax.experimental.pallas import tpu_sc as plsc

_B = 128
_N = 4096
_D = 32
_K = 256
_NH = 16
_W = 49
_L = 16
_MSB = -(2 ** 31)


def _srl(x, n):
    if isinstance(n, int):
        nv = jnp.full(x.shape, n, jnp.int32) if getattr(x, "shape", ()) else jnp.int32(n)
    else:
        nv = jnp.broadcast_to(n, x.shape).astype(jnp.int32) if getattr(x, "shape", ()) else n
    return lax.shift_right_logical(x, nv)


def _iota16():
    return lax.broadcasted_iota(jnp.int32, (_L,), 0)


def _meankey_body(qt_ref, uk_ref):
    blk = qt_ref[...]
    vs = [blk[:, d, :] for d in range(_D)]
    while len(vs) > 1:
        vs = [vs[i] + vs[i + 1] for i in range(0, len(vs), 2)]
    x = vs[0] * jnp.float32(1.0 / _D)
    u = lax.bitcast_convert_type(x, jnp.int32)
    uk_ref[...] = jnp.where(u >= 0, u ^ _MSB, ~u)


def _meankey(qt):
    BB = 8
    return pl.pallas_call(
        _meankey_body,
        grid=(_B // BB,),
        in_specs=[pl.BlockSpec((BB, _D, _N), lambda b: (b, 0, 0))],
        out_specs=pl.BlockSpec((BB, _N), lambda b: (b, 0)),
        out_shape=jax.ShapeDtypeStruct((_B, _N), jnp.int32),
    )(qt)


def _sc_select_body(uk_hbm, q2, qq2, pout, qal, val,
                    ukey, pidx, hist, gidx0, gidx1, qalb, valb,
                    sem, semg, semh):
    c = lax.axis_index("c")
    s = lax.axis_index("s")
    wid = s * 2 + c
    iota = _iota16()

    def row_body(j, _carry):
        b = wid * 4 + j
        pltpu.async_copy(uk_hbm.at[b], ukey, sem).wait()

        r = jnp.int32(_K)
        prefix = jnp.int32(0)
        for p in range(4):
            shift = 24 - 8 * p
            for g2 in range(16):
                hist[pl.ds(g2 * _L, _L)] = jnp.zeros((_L,), jnp.int32)
            if p > 0:
                phi = _srl(prefix, shift + 8)
                phiv = jnp.broadcast_to(phi, (_L,))

            def hist_body(i, _):
                for u2 in range(4):
                    uk = ukey[pl.ds((i * 4 + u2) * _L, _L)]
                    dig = jnp.bitwise_and(_srl(uk, shift),
                                          jnp.full((_L,), 0xFF, jnp.int32))
                    if p > 0:
                        m = _srl(uk, shift + 8) == phiv
                        cnt, lastm = plsc.scan_count(dig, mask=m)
                    else:
                        cnt, lastm = plsc.scan_count(dig)
                    plsc.addupdate_scatter(hist, [dig], cnt, mask=lastm)
                return 0

            lax.fori_loop(0, _N // (_L * 4), hist_body, 0)

            tot = jnp.int32(0)
            for g2 in range(16):
                tot = tot + jnp.sum(hist[pl.ds(g2 * _L, _L)])
            run = jnp.int32(0)
            cntge = jnp.zeros((_L,), jnp.int32)
            for g2 in range(16):
                h = hist[pl.ds(g2 * _L, _L)]
                cs = plsc.cumsum(h)
                excl = cs - h + run
                suf = tot - excl
                cntge = cntge + (suf >= r).astype(jnp.int32)
                run = run + jnp.sum(h)
            dstar = jnp.sum(cntge) - 1
            dspl = jnp.broadcast_to(dstar, (_L,))
            hd = jnp.max(plsc.load_gather(hist, [dspl]))
            acc2 = jnp.zeros((_L,), jnp.int32)
            for g2 in range(16):
                h = hist[pl.ds(g2 * _L, _L)]
                binid = iota + g2 * _L
                acc2 = acc2 + jnp.where(binid < dspl, h, 0)
            excl_d = jnp.sum(acc2)
            r = r - (tot - excl_d - hd)
            prefix = prefix | lax.shift_left(dstar, jnp.int32(shift))

        tspl = jnp.broadcast_to(prefix, (_L,))
        tskv = jnp.broadcast_to(prefix ^ _MSB, (_L,))
        msbv = jnp.full((_L,), _MSB, jnp.int32)

        def sel_body(i, carry):
            selc, eqc = carry
            for u2 in range(2):
                uk = ukey[pl.ds((i * 2 + u2) * _L, _L)]
                m_eq = uk == tspl
                m_gt = (uk ^ msbv) > tskv
                e32 = m_eq.astype(jnp.int32)
                eex = plsc.cumsum(e32) - e32
                take_eq = m_eq & ((eqc + eex) < r)
                sel = m_gt | take_eq
                s32v = sel.astype(jnp.int32)
                sex = plsc.cumsum(s32v) - s32v
                pos = selc + sex
                plsc.store_scatter(pidx, [pos], iota + (i * 2 + u2) * _L,
                                   mask=sel)
                selc = selc + jnp.sum(s32v)
                eqc = eqc + jnp.sum(e32)
            return (selc, eqc)

        lax.fori_loop(0, _N // (_L * 2), sel_body,
                      (jnp.int32(0), jnp.int32(0)))
        pltpu.sync_copy(pidx, pout.at[b])

        base_row = b * _N
        for h2 in range(2):
            gref = gidx0 if h2 == 0 else gidx1
            for t in range(8):
                off = h2 * 128 + t * _L
                gref[pl.ds(t * _L, _L)] = pidx[pl.ds(off, _L)] + base_row
        cp1 = pltpu.async_copy(qq2.at[gidx0], qalb.at[pl.ds(0, 128)], semg)
        cp2 = pltpu.async_copy(qq2.at[gidx1], qalb.at[pl.ds(128, 128)], semg)
        cp3 = pltpu.async_copy(q2.at[gidx0], valb.at[pl.ds(0, 128)], semh)
        cp4 = pltpu.async_copy(q2.at[gidx1], valb.at[pl.ds(128, 128)], semh)
        cp1.wait(); cp2.wait(); cp3.wait(); cp4.wait()
        pltpu.sync_copy(qalb, qal.at[b])
        pltpu.sync_copy(valb, val.at[b])
        return 0

    lax.fori_loop(0, 4, row_body, 0)


def _sc_yal_body(bias2, pin, yal, pbuf, brow, ybuf, semy):
    c = lax.axis_index("c")
    s = lax.axis_index("s")
    wid = s * 2 + c
    h = jnp.remainder(wid, _NH)
    half = wid // _NH
    w0 = half * 25
    nw = jnp.where(half == 0, 25, 24)
    for bh in range(8):
        pltpu.sync_copy(pin.at[bh * _NH + h], pbuf.at[bh])

    def w_body(wi, _):
        w = w0 + wi
        pltpu.sync_copy(bias2.at[h * _W + w], brow)
        for bh in range(8):
            for ch in range(_K // _L):
                idxv = pbuf[bh, pl.ds(ch * _L, _L)]
                ybuf[bh, pl.ds(ch * _L, _L)] = plsc.load_gather(brow, [idxv])
        for bh in range(8):
            pltpu.sync_copy(ybuf.at[bh], yal.at[bh, h, w])
        return 0

    lax.fori_loop(0, nw, w_body, 0)


def _make_sc_kernels():
    mesh = plsc.VectorSubcoreMesh(core_axis_name="c", subcore_axis_name="s")
    cparams = pltpu.CompilerParams(
        needs_layout_passes=False, use_tc_tiling_on_sc=False)
    sel = pl.kernel(
        _sc_select_body,
        out_type=(
            jax.ShapeDtypeStruct((_B, _K), jnp.int32),
            jax.ShapeDtypeStruct((_B, _K, _D), jnp.float32),
            jax.ShapeDtypeStruct((_B, _K, _D), jnp.float32),
        ),
        mesh=mesh,
        compiler_params=cparams,
        scratch_types=[
            pltpu.VMEM((_N,), jnp.int32),
            pltpu.VMEM((_K,), jnp.int32),
            pltpu.VMEM((256,), jnp.int32),
            pltpu.VMEM((128,), jnp.int32),
            pltpu.VMEM((128,), jnp.int32),
            pltpu.VMEM((_K, _D), jnp.float32),
            pltpu.VMEM((_K, _D), jnp.float32),
            pltpu.SemaphoreType.DMA,
            pltpu.SemaphoreType.DMA,
            pltpu.SemaphoreType.DMA,
        ],
    )
    yalk = pl.kernel(
        _sc_yal_body,
        out_type=jax.ShapeDtypeStruct((_B // _NH, _NH, _W, _K), jnp.float32),
        mesh=mesh,
        compiler_params=cparams,
        scratch_types=[
            pltpu.VMEM((8, _K), jnp.int32),
            pltpu.VMEM((_N,), jnp.float32),
            pltpu.VMEM((8, _K), jnp.float32),
            pltpu.SemaphoreType.DMA,
        ],
    )
    return sel, yalk


_SC_SELECT, _SC_YAL = _make_sc_kernels()


def kernel(q, qq, bias):
    qt = jnp.transpose(q, (0, 2, 1))
    uk = _meankey(qt)
    q2 = q.reshape(-1, _D)
    qq2 = qq.reshape(-1, _D)
    bias2 = bias.reshape(_NH * _W, _N)
    p, qal, val = _SC_SELECT(uk, q2, qq2)
    yal = _SC_YAL(bias2, p)
    return (qal, val, yal)


# speedup vs baseline: 1.0562x; 1.0562x over previous
import jax
import jax.numpy as jnp
from jax import lax
from jax.experimental import pallas as pl
from jax.experimental.pallas import tpu as pltpu
from jax.experimental.pallas import tpu_sc as plsc

_B = 128
_N = 4096
_D = 32
_K = 256
_NH = 16
_W = 49
_L = 16
_MSB = -(2 ** 31)


def _srl(x, n):
    if isinstance(n, int):
        nv = jnp.full(x.shape, n, jnp.int32) if getattr(x, "shape", ()) else jnp.int32(n)
    else:
        nv = jnp.broadcast_to(n, x.shape).astype(jnp.int32) if getattr(x, "shape", ()) else n
    return lax.shift_right_logical(x, nv)


def _iota16():
    return lax.broadcasted_iota(jnp.int32, (_L,), 0)


def _meankey_body(qt_ref, uk_ref):
    blk = qt_ref[...]
    vs = [blk[:, d, :] for d in range(_D)]
    while len(vs) > 1:
        vs = [vs[i] + vs[i + 1] for i in range(0, len(vs), 2)]
    x = vs[0] * jnp.float32(1.0 / _D)
    u = lax.bitcast_convert_type(x, jnp.int32)
    uk_ref[...] = jnp.where(u >= 0, u ^ _MSB, ~u)


def _meankey(qt):
    BB = 8
    return pl.pallas_call(
        _meankey_body,
        grid=(_B // BB,),
        in_specs=[pl.BlockSpec((BB, _D, _N), lambda b: (b, 0, 0))],
        out_specs=pl.BlockSpec((BB, _N), lambda b: (b, 0)),
        out_shape=jax.ShapeDtypeStruct((_B, _N), jnp.int32),
    )(qt)


def _sc_select_body(uk_hbm, pout, ukey, pidx, hist, sem):
    c = lax.axis_index("c")
    s = lax.axis_index("s")
    wid = s * 2 + c
    iota = _iota16()

    def row_body(j, _carry):
        b = wid * 4 + j
        pltpu.async_copy(uk_hbm.at[b], ukey, sem).wait()

        r = jnp.int32(_K)
        prefix = jnp.int32(0)
        for p in range(4):
            shift = 24 - 8 * p
            for g2 in range(16):
                hist[pl.ds(g2 * _L, _L)] = jnp.zeros((_L,), jnp.int32)
            if p > 0:
                phi = _srl(prefix, shift + 8)
                phiv = jnp.broadcast_to(phi, (_L,))

            def hist_body(i, _):
                for u2 in range(4):
                    uk = ukey[pl.ds((i * 4 + u2) * _L, _L)]
                    dig = jnp.bitwise_and(_srl(uk, shift),
                                          jnp.full((_L,), 0xFF, jnp.int32))
                    if p > 0:
                        m = _srl(uk, shift + 8) == phiv
                        cnt, lastm = plsc.scan_count(dig, mask=m)
                    else:
                        cnt, lastm = plsc.scan_count(dig)
                    plsc.addupdate_scatter(hist, [dig], cnt, mask=lastm)
                return 0

            lax.fori_loop(0, _N // (_L * 4), hist_body, 0)

            tot = jnp.int32(0)
            for g2 in range(16):
                tot = tot + jnp.sum(hist[pl.ds(g2 * _L, _L)])
            run = jnp.int32(0)
            cntge = jnp.zeros((_L,), jnp.int32)
            for g2 in range(16):
                h = hist[pl.ds(g2 * _L, _L)]
                cs = plsc.cumsum(h)
                excl = cs - h + run
                suf = tot - excl
                cntge = cntge + (suf >= r).astype(jnp.int32)
                run = run + jnp.sum(h)
            dstar = jnp.sum(cntge) - 1
            dspl = jnp.broadcast_to(dstar, (_L,))
            hd = jnp.max(plsc.load_gather(hist, [dspl]))
            acc2 = jnp.zeros((_L,), jnp.int32)
            for g2 in range(16):
                h = hist[pl.ds(g2 * _L, _L)]
                binid = iota + g2 * _L
                acc2 = acc2 + jnp.where(binid < dspl, h, 0)
            excl_d = jnp.sum(acc2)
            r = r - (tot - excl_d - hd)
            prefix = prefix | lax.shift_left(dstar, jnp.int32(shift))

        tspl = jnp.broadcast_to(prefix, (_L,))
        tskv = jnp.broadcast_to(prefix ^ _MSB, (_L,))
        msbv = jnp.full((_L,), _MSB, jnp.int32)

        def sel_body(i, carry):
            selc, eqc = carry
            for u2 in range(2):
                uk = ukey[pl.ds((i * 2 + u2) * _L, _L)]
                m_eq = uk == tspl
                m_gt = (uk ^ msbv) > tskv
                e32 = m_eq.astype(jnp.int32)
                eex = plsc.cumsum(e32) - e32
                take_eq = m_eq & ((eqc + eex) < r)
                sel = m_gt | take_eq
                s32v = sel.astype(jnp.int32)
                sex = plsc.cumsum(s32v) - s32v
                pos = selc + sex
                plsc.store_scatter(pidx, [pos], iota + (i * 2 + u2) * _L,
                                   mask=sel)
                selc = selc + jnp.sum(s32v)
                eqc = eqc + jnp.sum(e32)
            return (selc, eqc)

        lax.fori_loop(0, _N // (_L * 2), sel_body,
                      (jnp.int32(0), jnp.int32(0)))
        pltpu.sync_copy(pidx, pout.at[b])
        return 0

    lax.fori_loop(0, 4, row_body, 0)


def _sc_rowgather_body(q2, qq2, pin, qal, val,
                       pidx, gidx0, gidx1, qalb, valb, semp, semg, semh):
    c = lax.axis_index("c")
    s = lax.axis_index("s")
    wid = s * 2 + c

    def row_body(j, _carry):
        b = wid * 4 + j
        base_row = b * _N
        pltpu.async_copy(pin.at[b], pidx, semp).wait()
        for h2 in range(2):
            gref = gidx0 if h2 == 0 else gidx1
            for t in range(8):
                off = h2 * 128 + t * _L
                gref[pl.ds(t * _L, _L)] = pidx[pl.ds(off, _L)] + base_row
        cp1 = pltpu.async_copy(qq2.at[gidx0], qalb.at[pl.ds(0, 128)], semg)
        cp2 = pltpu.async_copy(qq2.at[gidx1], qalb.at[pl.ds(128, 128)], semg)
        cp3 = pltpu.async_copy(q2.at[gidx0], valb.at[pl.ds(0, 128)], semh)
        cp4 = pltpu.async_copy(q2.at[gidx1], valb.at[pl.ds(128, 128)], semh)
        cp1.wait(); cp2.wait(); cp3.wait(); cp4.wait()
        pltpu.sync_copy(qalb, qal.at[b])
        pltpu.sync_copy(valb, val.at[b])
        return 0

    lax.fori_loop(0, 4, row_body, 0)


def _sc_yal_body(bias2, pin, yal, pbuf, brow, ybuf, semy):
    c = lax.axis_index("c")
    s = lax.axis_index("s")
    wid = s * 2 + c
    h = jnp.remainder(wid, _NH)
    half = wid // _NH
    w0 = half * 25
    nw = jnp.where(half == 0, 25, 24)
    for bh in range(8):
        pltpu.sync_copy(pin.at[bh * _NH + h], pbuf.at[bh])

    def w_body(wi, _):
        w = w0 + wi
        pltpu.sync_copy(bias2.at[h * _W + w], brow)
        for bh in range(8):
            for ch in range(_K // _L):
                idxv = pbuf[bh, pl.ds(ch * _L, _L)]
                ybuf[bh, pl.ds(ch * _L, _L)] = plsc.load_gather(brow, [idxv])
        for bh in range(8):
            pltpu.sync_copy(ybuf.at[bh], yal.at[bh, h, w])
        return 0

    lax.fori_loop(0, nw, w_body, 0)


def _make_sc_kernels():
    mesh = plsc.VectorSubcoreMesh(core_axis_name="c", subcore_axis_name="s")
    cparams = pltpu.CompilerParams(
        needs_layout_passes=False, use_tc_tiling_on_sc=False)
    sel = pl.kernel(
        _sc_select_body,
        out_type=jax.ShapeDtypeStruct((_B, _K), jnp.int32),
        mesh=mesh,
        compiler_params=cparams,
        scratch_types=[
            pltpu.VMEM((_N,), jnp.int32),
            pltpu.VMEM((_K,), jnp.int32),
            pltpu.VMEM((256,), jnp.int32),
            pltpu.SemaphoreType.DMA,
        ],
    )
    rowg = pl.kernel(
        _sc_rowgather_body,
        out_type=(
            jax.ShapeDtypeStruct((_B, _K, _D), jnp.float32),
            jax.ShapeDtypeStruct((_B, _K, _D), jnp.float32),
        ),
        mesh=mesh,
        compiler_params=cparams,
        scratch_types=[
            pltpu.VMEM((_K,), jnp.int32),
            pltpu.VMEM((128,), jnp.int32),
            pltpu.VMEM((128,), jnp.int32),
            pltpu.VMEM((_K, _D), jnp.float32),
            pltpu.VMEM((_K, _D), jnp.float32),
            pltpu.SemaphoreType.DMA,
            pltpu.SemaphoreType.DMA,
            pltpu.SemaphoreType.DMA,
        ],
    )
    yalk = pl.kernel(
        _sc_yal_body,
        out_type=jax.ShapeDtypeStruct((_B // _NH, _NH, _W, _K), jnp.float32),
        mesh=mesh,
        compiler_params=cparams,
        scratch_types=[
            pltpu.VMEM((8, _K), jnp.int32),
            pltpu.VMEM((_N,), jnp.float32),
            pltpu.VMEM((8, _K), jnp.float32),
            pltpu.SemaphoreType.DMA,
        ],
    )
    return sel, rowg, yalk


_SC_SELECT, _SC_ROWGATHER, _SC_YAL = _make_sc_kernels()


def kernel(q, qq, bias):
    qt = jnp.transpose(q, (0, 2, 1))
    uk = _meankey(qt)
    p = _SC_SELECT(uk)
    q2 = q.reshape(-1, _D)
    qq2 = qq.reshape(-1, _D)
    bias2 = bias.reshape(_NH * _W, _N)
    qal, val = _SC_ROWGATHER(q2, qq2, p)
    yal = _SC_YAL(bias2, p)
    return (qal, val, yal)
